# baseline (speedup 1.0000x reference)
import functools

import jax
import jax.numpy as jnp
from jax.experimental import pallas as pl
from jax.experimental.pallas import tpu as pltpu

_C_BLK = 1024
_TEMPERATURE = 1.5


def _retrieve_body(s_ref, xt_ref, mu_ref, std_ref, o_ref, *, min_std):
    acc = jax.lax.dot_general(
        s_ref[...],
        xt_ref[...],
        (((1,), (0,)), ((), ())),
        preferred_element_type=jnp.float32,
    )
    z = (acc - mu_ref[...]) / jnp.maximum(std_ref[...], min_std)
    z = jnp.nan_to_num(z, nan=0.0, posinf=10.0, neginf=-10.0)
    o_ref[...] = z / _TEMPERATURE


def kernel(x_bits, S, z_mu, z_std):
    B, N = x_bits.shape
    C = S.shape[0]
    xt_f8 = x_bits.T.astype(jnp.float4_e2m1fn)
    s_f8 = S.astype(jnp.float4_e2m1fn)
    mu2 = z_mu.reshape(C, 1)
    std2 = z_std.reshape(C, 1)
    min_std = max(1e-6, 1.0 / (B**0.5)) if B > 0 else 1e-6
    out_t = pl.pallas_call(
        functools.partial(_retrieve_body, min_std=min_std),
        grid=(pl.cdiv(C, _C_BLK),),
        in_specs=[
            pl.BlockSpec((_C_BLK, N), lambda i: (i, 0)),
            pl.BlockSpec((N, B), lambda i: (0, 0)),
            pl.BlockSpec((_C_BLK, 1), lambda i: (i, 0)),
            pl.BlockSpec((_C_BLK, 1), lambda i: (i, 0)),
        ],
        out_specs=pl.BlockSpec((_C_BLK, B), lambda i: (i, 0)),
        out_shape=jax.ShapeDtypeStruct((C, B), jnp.float32),
        compiler_params=pltpu.CompilerParams(
            dimension_semantics=("arbitrary",),
            allow_input_fusion=[True, False, False, False],
        ),
    )(s_f8, xt_f8, mu2, std2)
    return out_t.T


# speedup vs baseline: 1.0593x; 1.0593x over previous
import functools

import jax
import jax.numpy as jnp
from jax.experimental import pallas as pl
from jax.experimental.pallas import tpu as pltpu

_C_BLK = 2048
_TEMPERATURE = 1.5


def _retrieve_body(s_ref, xt_ref, mu_ref, std_ref, o_ref, *, min_std):
    acc = jax.lax.dot_general(
        s_ref[...],
        xt_ref[...],
        (((1,), (0,)), ((), ())),
        preferred_element_type=jnp.float32,
    )
    z = (acc - mu_ref[...]) / jnp.maximum(std_ref[...], min_std)
    z = jnp.nan_to_num(z, nan=0.0, posinf=10.0, neginf=-10.0)
    o_ref[...] = z / _TEMPERATURE


def kernel(x_bits, S, z_mu, z_std):
    B, N = x_bits.shape
    C = S.shape[0]
    xt_f8 = x_bits.T.astype(jnp.float8_e4m3fn)
    s_f8 = S.astype(jnp.float8_e4m3fn)
    mu2 = z_mu.reshape(C, 1)
    std2 = z_std.reshape(C, 1)
    min_std = max(1e-6, 1.0 / (B**0.5)) if B > 0 else 1e-6
    out_t = pl.pallas_call(
        functools.partial(_retrieve_body, min_std=min_std),
        grid=(pl.cdiv(C, _C_BLK),),
        in_specs=[
            pl.BlockSpec((_C_BLK, N), lambda i: (i, 0)),
            pl.BlockSpec((N, B), lambda i: (0, 0)),
            pl.BlockSpec((_C_BLK, 1), lambda i: (i, 0)),
            pl.BlockSpec((_C_BLK, 1), lambda i: (i, 0)),
        ],
        out_specs=pl.BlockSpec((_C_BLK, B), lambda i: (i, 0)),
        out_shape=jax.ShapeDtypeStruct((C, B), jnp.float32),
        compiler_params=pltpu.CompilerParams(
            dimension_semantics=("arbitrary",),
            allow_input_fusion=[True, False, False, False],
        ),
    )(s_f8, xt_f8, mu2, std2)
    return out_t.T


# speedup vs baseline: 1.0907x; 1.0296x over previous
import functools

import jax
import jax.numpy as jnp
from jax.experimental import pallas as pl
from jax.experimental.pallas import tpu as pltpu

_C_BLK = 512
_TEMPERATURE = 1.5


def _retrieve_body(s_ref, xt_ref, mu_ref, std_ref, o_ref, *, min_std):
    acc = jax.lax.dot_general(
        s_ref[...],
        xt_ref[...],
        (((1,), (0,)), ((), ())),
        preferred_element_type=jnp.float32,
    )
    z = (acc - mu_ref[...]) / jnp.maximum(std_ref[...], min_std)
    z = jnp.nan_to_num(z, nan=0.0, posinf=10.0, neginf=-10.0)
    o_ref[...] = z / _TEMPERATURE


def kernel(x_bits, S, z_mu, z_std):
    B, N = x_bits.shape
    C = S.shape[0]
    xt_f8 = x_bits.T.astype(jnp.float8_e4m3fn)
    s_f8 = S.astype(jnp.float8_e4m3fn)
    mu2 = z_mu.reshape(C, 1)
    std2 = z_std.reshape(C, 1)
    min_std = max(1e-6, 1.0 / (B**0.5)) if B > 0 else 1e-6
    out_t = pl.pallas_call(
        functools.partial(_retrieve_body, min_std=min_std),
        grid=(pl.cdiv(C, _C_BLK),),
        in_specs=[
            pl.BlockSpec((_C_BLK, N), lambda i: (i, 0)),
            pl.BlockSpec((N, B), lambda i: (0, 0)),
            pl.BlockSpec((_C_BLK, 1), lambda i: (i, 0)),
            pl.BlockSpec((_C_BLK, 1), lambda i: (i, 0)),
        ],
        out_specs=pl.BlockSpec((_C_BLK, B), lambda i: (i, 0)),
        out_shape=jax.ShapeDtypeStruct((C, B), jnp.float32),
        compiler_params=pltpu.CompilerParams(
            dimension_semantics=("arbitrary",),
            allow_input_fusion=[True, False, False, False],
        ),
    )(s_f8, xt_f8, mu2, std2)
    return out_t.T


# speedup vs baseline: 1.3559x; 1.2431x over previous
import functools

import jax
import jax.numpy as jnp
from jax.experimental import pallas as pl
from jax.experimental.pallas import tpu as pltpu

_C_BLK = 1024
_TEMPERATURE = 1.5


def _retrieve_body(x_ref, s_ref, mu_ref, std_ref, o_ref, *, min_std):
    acc = jax.lax.dot_general(
        x_ref[...],
        s_ref[...],
        (((1,), (1,)), ((), ())),
        preferred_element_type=jnp.float32,
    )
    z = (acc - mu_ref[...]) / jnp.maximum(std_ref[...], min_std)
    z = jnp.nan_to_num(z, nan=0.0, posinf=10.0, neginf=-10.0)
    o_ref[...] = z / _TEMPERATURE


def kernel(x_bits, S, z_mu, z_std):
    B, N = x_bits.shape
    C = S.shape[0]
    x_f8 = x_bits.astype(jnp.float8_e4m3fn)
    s_f8 = S.astype(jnp.float8_e4m3fn)
    mu2 = z_mu.reshape(1, C)
    std2 = z_std.reshape(1, C)
    min_std = max(1e-6, 1.0 / (B**0.5)) if B > 0 else 1e-6
    return pl.pallas_call(
        functools.partial(_retrieve_body, min_std=min_std),
        grid=(pl.cdiv(C, _C_BLK),),
        in_specs=[
            pl.BlockSpec((B, N), lambda i: (0, 0)),
            pl.BlockSpec((_C_BLK, N), lambda i: (i, 0)),
            pl.BlockSpec((1, _C_BLK), lambda i: (0, i)),
            pl.BlockSpec((1, _C_BLK), lambda i: (0, i)),
        ],
        out_specs=pl.BlockSpec((B, _C_BLK), lambda i: (0, i)),
        out_shape=jax.ShapeDtypeStruct((B, C), jnp.float32),
        compiler_params=pltpu.CompilerParams(
            dimension_semantics=("arbitrary",),
        ),
    )(x_f8, s_f8, mu2, std2)


# speedup vs baseline: 1.3624x; 1.0048x over previous
import functools

import jax
import jax.numpy as jnp
from jax.experimental import pallas as pl
from jax.experimental.pallas import tpu as pltpu

_C_BLK = 2048
_TEMPERATURE = 1.5


def _retrieve_body(x_ref, s_ref, mu_ref, std_ref, o_ref, *, min_std):
    acc = jax.lax.dot_general(
        x_ref[...],
        s_ref[...],
        (((1,), (1,)), ((), ())),
        preferred_element_type=jnp.float32,
    )
    z = (acc - mu_ref[...]) / jnp.maximum(std_ref[...], min_std)
    z = jnp.nan_to_num(z, nan=0.0, posinf=10.0, neginf=-10.0)
    o_ref[...] = z / _TEMPERATURE


def kernel(x_bits, S, z_mu, z_std):
    B, N = x_bits.shape
    C = S.shape[0]
    x_f8 = x_bits.astype(jnp.float8_e4m3fn)
    s_f8 = S.astype(jnp.float8_e4m3fn)
    mu2 = z_mu.reshape(1, C)
    std2 = z_std.reshape(1, C)
    min_std = max(1e-6, 1.0 / (B**0.5)) if B > 0 else 1e-6
    return pl.pallas_call(
        functools.partial(_retrieve_body, min_std=min_std),
        grid=(pl.cdiv(C, _C_BLK),),
        in_specs=[
            pl.BlockSpec((B, N), lambda i: (0, 0)),
            pl.BlockSpec((_C_BLK, N), lambda i: (i, 0)),
            pl.BlockSpec((1, _C_BLK), lambda i: (0, i)),
            pl.BlockSpec((1, _C_BLK), lambda i: (0, i)),
        ],
        out_specs=pl.BlockSpec((B, _C_BLK), lambda i: (0, i)),
        out_shape=jax.ShapeDtypeStruct((B, C), jnp.float32),
        compiler_params=pltpu.CompilerParams(
            dimension_semantics=("arbitrary",),
        ),
    )(x_f8, s_f8, mu2, std2)
